# Initial kernel scaffold; baseline (speedup 1.0000x reference)
import jax
import jax.numpy as jnp
from jax.experimental import pallas as pl


def kernel(h, x, vel, edge_attr, params, edges):
    raise NotImplementedError("write your pallas kernel here")


# speedup vs baseline: 3.1501x; 3.1501x over previous
import functools

import jax
import jax.numpy as jnp
from jax import lax
from jax.experimental import pallas as pl
from jax.experimental.pallas import tpu as pltpu
from jax.experimental.pallas import tpu_sc as plsc

F32 = jnp.float32

H = 128
XP = 8
COORDS_WEIGHT = 1.0

NC = 2
NS = 16
NW = NC * NS

C = 80

NP_PAD = 10240


def _silu(v):
    return v * (1.0 / (1.0 + jnp.exp(-v)))


def _lane3():
    return (lax.broadcasted_iota(jnp.int32, (1, XP), 1) == 3).astype(F32)


def _mask012():
    return (lax.broadcasted_iota(jnp.int32, (1, XP), 1) < 3).astype(F32)


def _sc_gather(A, B, xp, row3, col3, E):
    N = xp.shape[0] // XP
    epw = E // NW
    nch = epw // C
    ngrp = C // 16
    mesh = plsc.VectorSubcoreMesh(core_axis_name="c", subcore_axis_name="s")
    out_type = (
        jax.ShapeDtypeStruct((E, H), F32),
        jax.ShapeDtypeStruct((E, H), F32),
        jax.ShapeDtypeStruct((E * XP,), F32),
    )
    scratch = [
        pltpu.VMEM((N * XP,), F32),
        pltpu.VMEM((C,), jnp.int32),
        pltpu.VMEM((C,), jnp.int32),
        pltpu.VMEM((C, H), F32),
        pltpu.VMEM((C, H), F32),
        pltpu.VMEM((C * XP,), F32),
        pltpu.SemaphoreType.DMA,
        pltpu.SemaphoreType.DMA,
    ]

    def body(a_h, b_h, x_h, row_h, col_h, ag_h, bg_h, xd_h,
             xpv, rowc, colc, bufa, bufb, bufd, gsem, wsem):
        wid = lax.axis_index("s") * NC + lax.axis_index("c")
        pltpu.sync_copy(x_h, xpv)
        iota = lax.iota(jnp.int32, 16)
        zf = jnp.zeros((16,), F32)
        for k in range(ngrp):
            eidx8 = (iota + k * 16) * XP
            for l in range(4, XP):
                plsc.store_scatter(bufd, [eidx8 + l], zf)

        @pl.loop(0, nch)
        def _chunk(j):
            base = wid * epw + j * C
            pltpu.sync_copy(row_h.at[wid, j], rowc)
            pltpu.sync_copy(col_h.at[wid, j], colc)
            d1 = pltpu.async_copy(a_h.at[rowc], bufa, gsem)
            d2 = pltpu.async_copy(b_h.at[colc], bufb, gsem)
            for k in range(ngrp):
                ridx8 = rowc[pl.ds(k * 16, 16)] * XP
                cidx8 = colc[pl.ds(k * 16, 16)] * XP
                eidx8 = (iota + k * 16) * XP
                rad = jnp.zeros((16,), F32)
                for l in range(3):
                    dl = (plsc.load_gather(xpv, [ridx8 + l])
                          - plsc.load_gather(xpv, [cidx8 + l]))
                    plsc.store_scatter(bufd, [eidx8 + l], dl)
                    rad = rad + dl * dl
                plsc.store_scatter(bufd, [eidx8 + 3], rad)
            d1.wait()
            d2.wait()
            w1 = pltpu.async_copy(bufa, ag_h.at[pl.ds(base, C)], wsem)
            w2 = pltpu.async_copy(bufb, bg_h.at[pl.ds(base, C)], wsem)
            w3 = pltpu.async_copy(bufd, xd_h.at[pl.ds(base * XP, C * XP)], wsem)
            w1.wait()
            w2.wait()
            w3.wait()

    return pl.kernel(body, out_type=out_type, mesh=mesh,
                     scratch_types=scratch,
                     compiler_params=pltpu.CompilerParams(
                         needs_layout_passes=False),
                     )(A, B, xp, row3, col3)


def _sc_scatter(m, tr, row3, zh, zx, E):
    epw = E // NW
    nch = epw // C
    rows_per_tile = NP_PAD // NS
    xwords_per_tile = NP_PAD * XP // NS
    nxs = C * XP // 128
    mesh = plsc.VectorSubcoreMesh(core_axis_name="c", subcore_axis_name="s")
    out_type = (
        jax.ShapeDtypeStruct((NC, NP_PAD, H), F32),
        jax.ShapeDtypeStruct((NC, NP_PAD * XP), F32),
    )
    scratch = [
        pltpu.VMEM((C,), jnp.int32),
        pltpu.VMEM((C, H), F32),
        pltpu.VMEM((C * XP,), F32),
        pltpu.VMEM((nxs, 128), jnp.int32),
        pltpu.VMEM_SHARED((NP_PAD, H), F32),
        pltpu.VMEM_SHARED((NP_PAD * XP,), F32),
        pltpu.SemaphoreType.DMA,
        pltpu.SemaphoreType.DMA,
    ]

    def body(m_h, tr_h, row_h, zh_h, zx_h, aggh_h, aggx_h,
             rowc, bufm, buft, idxb, sh, sx, lsem, asem):
        cid = lax.axis_index("c")
        sid = lax.axis_index("s")
        wid = sid * NC + cid
        r0 = sid * rows_per_tile
        x0 = sid * xwords_per_tile
        pltpu.sync_copy(zx_h, sx.at[pl.ds(x0, xwords_per_tile)])
        pltpu.sync_copy(zh_h, sh.at[pl.ds(r0, rows_per_tile)])
        plsc.subcore_barrier()
        iota = lax.iota(jnp.int32, 16)
        imod = jnp.bitwise_and(iota, XP - 1)
        ediv = jnp.right_shift(iota, 3)

        @pl.loop(0, nch)
        def _chunk(j):
            base = wid * epw + j * C
            d1 = pltpu.async_copy(m_h.at[pl.ds(base, C)], bufm, lsem)
            d2 = pltpu.async_copy(tr_h.at[pl.ds(base * XP, C * XP)], buft, lsem)
            d3 = pltpu.async_copy(row_h.at[wid, j], rowc, lsem)
            d1.wait()
            d2.wait()
            d3.wait()
            dm = pltpu.async_copy(bufm, sh.at[rowc], asem, add=True)
            for s in range(nxs):
                for v in range(8):
                    e0 = s * 16 + 2 * v
                    rv = plsc.load_gather(rowc, [e0 + ediv])
                    idxb[s, pl.ds(v * 16, 16)] = rv * XP + imod
            for s in range(nxs):
                pltpu.sync_copy(buft.at[pl.ds(s * 128, 128)],
                                sx.at[idxb.at[s]], add=True)
            dm.wait()

        plsc.subcore_barrier()
        pltpu.sync_copy(sh.at[pl.ds(r0, rows_per_tile)],
                        aggh_h.at[cid, pl.ds(r0, rows_per_tile)])
        pltpu.sync_copy(sx.at[pl.ds(x0, xwords_per_tile)],
                        aggx_h.at[cid, pl.ds(x0, xwords_per_tile)])

    return pl.kernel(body, out_type=out_type, mesh=mesh,
                     scratch_types=scratch,
                     compiler_params=pltpu.CompilerParams(
                         needs_layout_passes=False),
                     )(m, tr, row3, zh, zx)


def _full_spec(shape):
    nd = len(shape)
    return pl.BlockSpec(shape, lambda i: (0,) * nd)


def _edge_block(E):
    return max(b for b in range(8, min(2560, E) + 1, 8) if E % b == 0)


def _tc_edge(ag, bg, xd, ea, wr, we1e, we2, be2, wc1, bc1, wc2p):
    E = ag.shape[0]
    BE = _edge_block(E)
    DE = ea.shape[1]

    def body(ag_r, bg_r, xd_r, ea_r, wr_r, we1e_r, we2_r, be2_r,
             wc1_r, bc1_r, wc2_r, m_r, tr_r):
        lane3 = _lane3()
        mask012 = _mask012()
        d = xd_r[...] * mask012
        radial = jnp.sum(xd_r[...] * lane3, axis=1, keepdims=True)
        pre = (ag_r[...] + bg_r[...] + radial * wr_r[...]
               + jnp.dot(ea_r[...], we1e_r[...], preferred_element_type=F32))
        m1 = _silu(pre)
        m2 = _silu(jnp.dot(m1, we2_r[...], preferred_element_type=F32)
                   + be2_r[...])
        u = _silu(jnp.dot(m2, wc1_r[...], preferred_element_type=F32)
                  + bc1_r[...])
        w8 = jnp.dot(u, wc2_r[...], preferred_element_type=F32)
        w = jnp.sum(w8, axis=1, keepdims=True)
        m_r[...] = m2
        tr_r[...] = d * w + lane3

    return pl.pallas_call(
        body,
        grid=(E // BE,),
        in_specs=[
            pl.BlockSpec((BE, H), lambda i: (i, 0)),
            pl.BlockSpec((BE, H), lambda i: (i, 0)),
            pl.BlockSpec((BE, XP), lambda i: (i, 0)),
            pl.BlockSpec((BE, DE), lambda i: (i, 0)),
            _full_spec((1, H)),
            _full_spec((DE, H)),
            _full_spec((H, H)),
            _full_spec((1, H)),
            _full_spec((H, H)),
            _full_spec((1, H)),
            _full_spec((H, XP)),
        ],
        out_specs=[
            pl.BlockSpec((BE, H), lambda i: (i, 0)),
            pl.BlockSpec((BE, XP), lambda i: (i, 0)),
        ],
        out_shape=[
            jax.ShapeDtypeStruct((E, H), F32),
            jax.ShapeDtypeStruct((E, XP), F32),
        ],
        compiler_params=pltpu.CompilerParams(
            dimension_semantics=("arbitrary",)),
    )(ag, bg, xd, ea, wr, we1e, we2, be2, wc1, bc1, wc2p)


def _node_block(N):
    return max(b for b in range(8, min(2048, N) + 1, 8) if N % b == 0)


def _tc_embed(h, wemb, bemb, we1r, be1, we1c):
    N, DIN = h.shape
    BN = _node_block(N)

    def body(h_r, wemb_r, bemb_r, we1r_r, be1_r, we1c_r, h0_r, a_r, b_r):
        h0 = jnp.dot(h_r[...], wemb_r[...], preferred_element_type=F32) \
            + bemb_r[...]
        h0_r[...] = h0
        a_r[...] = jnp.dot(h0, we1r_r[...], preferred_element_type=F32) \
            + be1_r[...]
        b_r[...] = jnp.dot(h0, we1c_r[...], preferred_element_type=F32)

    return pl.pallas_call(
        body,
        grid=(N // BN,),
        in_specs=[
            pl.BlockSpec((BN, DIN), lambda i: (i, 0)),
            _full_spec((DIN, H)),
            _full_spec((1, H)),
            _full_spec((H, H)),
            _full_spec((1, H)),
            _full_spec((H, H)),
        ],
        out_specs=[
            pl.BlockSpec((BN, H), lambda i: (i, 0)),
            pl.BlockSpec((BN, H), lambda i: (i, 0)),
            pl.BlockSpec((BN, H), lambda i: (i, 0)),
        ],
        out_shape=[
            jax.ShapeDtypeStruct((N, H), F32),
            jax.ShapeDtypeStruct((N, H), F32),
            jax.ShapeDtypeStruct((N, H), F32),
        ],
        compiler_params=pltpu.CompilerParams(
            dimension_semantics=("arbitrary",)),
    )(h, wemb, bemb, we1r, be1, we1c)


def _tc_node(h, xp, velp, aggh, aggx, wn1h, wn1a, bn1, wn2, bn2,
             wv1, bv1, wv2p, bv2p, we1rn, be1n, we1cn):
    N = h.shape[0]
    BN = _node_block(N)
    def body(h_r, xp_r, velp_r, aggh_r, aggx_r, wn1h_r, wn1a_r, bn1_r,
             wn2_r, bn2_r, wv1_r, bv1_r, wv2_r, bv2_r, we1rn_r, be1n_r,
             we1cn_r, ho_r, xo_r, a_r, b_r):
        lane3 = _lane3()
        mask012 = _mask012()
        hcur = h_r[...]
        ah = aggh_r[0] + aggh_r[1]
        ax = jnp.sum(aggx_r[...], axis=0)
        cnt = jnp.sum(ax * lane3, axis=1, keepdims=True)
        inv = COORDS_WEIGHT / jnp.maximum(cnt, 1.0)
        pv8 = jnp.dot(_silu(jnp.dot(hcur, wv1_r[...],
                                    preferred_element_type=F32) + bv1_r[...]),
                      wv2_r[...], preferred_element_type=F32) + bv2_r[...]
        phiv = jnp.sum(pv8, axis=1, keepdims=True)
        xo_r[...] = xp_r[...] + ax * mask012 * inv + phiv * velp_r[...]
        t = _silu(jnp.dot(hcur, wn1h_r[...], preferred_element_type=F32)
                  + jnp.dot(ah, wn1a_r[...], preferred_element_type=F32)
                  + bn1_r[...])
        hnew = hcur + jnp.dot(t, wn2_r[...], preferred_element_type=F32) \
            + bn2_r[...]
        ho_r[...] = hnew
        a_r[...] = jnp.dot(hnew, we1rn_r[...], preferred_element_type=F32) \
            + be1n_r[...]
        b_r[...] = jnp.dot(hnew, we1cn_r[...], preferred_element_type=F32)

    return pl.pallas_call(
        body,
        grid=(N // BN,),
        in_specs=[
            pl.BlockSpec((BN, H), lambda i: (i, 0)),
            pl.BlockSpec((BN, XP), lambda i: (i, 0)),
            pl.BlockSpec((BN, XP), lambda i: (i, 0)),
            pl.BlockSpec((NC, BN, H), lambda i: (0, i, 0)),
            pl.BlockSpec((NC, BN, XP), lambda i: (0, i, 0)),
            _full_spec((H, H)),
            _full_spec((H, H)),
            _full_spec((1, H)),
            _full_spec((H, H)),
            _full_spec((1, H)),
            _full_spec((H, H)),
            _full_spec((1, H)),
            _full_spec((H, XP)),
            _full_spec((1, XP)),
            _full_spec((H, H)),
            _full_spec((1, H)),
            _full_spec((H, H)),
        ],
        out_specs=[
            pl.BlockSpec((BN, H), lambda i: (i, 0)),
            pl.BlockSpec((BN, XP), lambda i: (i, 0)),
            pl.BlockSpec((BN, H), lambda i: (i, 0)),
            pl.BlockSpec((BN, H), lambda i: (i, 0)),
        ],
        out_shape=[
            jax.ShapeDtypeStruct((N, H), F32),
            jax.ShapeDtypeStruct((N, XP), F32),
            jax.ShapeDtypeStruct((N, H), F32),
            jax.ShapeDtypeStruct((N, H), F32),
        ],
        compiler_params=pltpu.CompilerParams(
            dimension_semantics=("arbitrary",)),
    )(h, xp, velp, aggh, aggx, wn1h, wn1a, bn1, wn2, bn2,
      wv1, bv1, wv2p, bv2p, we1rn, be1n, we1cn)


def _row2(v):
    return v.reshape(1, -1).astype(F32)


def _pad_minor(w, width):
    return jnp.pad(w.astype(F32), ((0, 0), (0, width - w.shape[1])))


def kernel(h, x, vel, edge_attr, params, edges):
    N = h.shape[0]
    E = edges.shape[1]
    row = edges[0]
    col = edges[1]
    epw = E // NW
    row3 = row.reshape(NW, epw // C, C)
    col3 = col.reshape(NW, epw // C, C)

    xp = jnp.pad(x.astype(F32), ((0, 0), (0, XP - x.shape[1])))
    velp = jnp.pad(vel.astype(F32), ((0, 0), (0, XP - vel.shape[1])))
    zh = jnp.zeros((NP_PAD // NS, H), F32)
    zx = jnp.zeros((NP_PAD * XP // NS,), F32)

    wemb, bemb = params["emb"]
    lp = params["layers"]
    n_layers = len(lp)

    def we1_parts(p):
        we1 = p["We1"]
        return (we1[:H], we1[H:2 * H], _row2(we1[2 * H]),
                we1[2 * H + 1:], _row2(p["be1"]))

    h0, A, B = _tc_embed(h, wemb, _row2(bemb), *(
        lambda t: (t[0], t[4], t[1]))(we1_parts(lp[0])))

    hc, xc = h0, xp
    for l in range(n_layers):
        p = lp[l]
        _, _, wr, we1e, _ = we1_parts(p)
        ag, bg, xdf = _sc_gather(A, B, xc.reshape(-1), row3, col3, E)
        m, tr = _tc_edge(ag, bg, xdf.reshape(E, XP), edge_attr, wr, we1e,
                         p["We2"], _row2(p["be2"]), p["Wc1"],
                         _row2(p["bc1"]), _pad_minor(p["Wc2"], XP))
        aggh, aggxf = _sc_scatter(m, tr.reshape(-1), row3, zh, zx, E)
        aggx = aggxf.reshape(NC, NP_PAD, XP)
        pn = lp[(l + 1) % n_layers]
        we1rn, we1cn, _, _, be1n = we1_parts(pn)
        wn1 = p["Wn1"]
        hc, xc, A, B = _tc_node(
            hc, xc, velp, aggh, aggx,
            wn1[:H], wn1[H:], _row2(p["bn1"]), p["Wn2"], _row2(p["bn2"]),
            p["Wv1"], _row2(p["bv1"]), _pad_minor(p["Wv2"], XP),
            _pad_minor(p["bv2"].reshape(1, 1), XP),
            we1rn, be1n, we1cn)

    return xc[:, :x.shape[1]]


# speedup vs baseline: 3.5280x; 1.1200x over previous
import functools

import jax
import jax.numpy as jnp
from jax import lax
from jax.experimental import pallas as pl
from jax.experimental.pallas import tpu as pltpu
from jax.experimental.pallas import tpu_sc as plsc

F32 = jnp.float32

H = 128
XP = 4
COORDS_WEIGHT = 1.0

NC = 2
NS = 16
NW = NC * NS

C = 80

NP_PAD = 10240


def _silu(v):
    return v * (1.0 / (1.0 + jnp.exp(-v)))


def _lane3():
    return (lax.broadcasted_iota(jnp.int32, (1, XP), 1) == 3).astype(F32)


def _mask012():
    return (lax.broadcasted_iota(jnp.int32, (1, XP), 1) < 3).astype(F32)


def _sc_gather(A, B, xp, row, col, E):
    N = xp.shape[0] // XP
    epw = E // NW
    nch = epw // C
    ngrp = C // 16
    mesh = plsc.VectorSubcoreMesh(core_axis_name="c", subcore_axis_name="s")
    out_type = (
        jax.ShapeDtypeStruct((E, H), F32),
        jax.ShapeDtypeStruct((E, H), F32),
        jax.ShapeDtypeStruct((E * XP,), F32),
    )
    scratch = [
        pltpu.VMEM((N * XP,), F32),
        pltpu.VMEM((C,), jnp.int32),
        pltpu.VMEM((C,), jnp.int32),
        pltpu.VMEM((C,), jnp.int32),
        pltpu.VMEM((C,), jnp.int32),
        pltpu.VMEM((C, H), F32),
        pltpu.VMEM((C, H), F32),
        pltpu.VMEM((C, H), F32),
        pltpu.VMEM((C, H), F32),
        pltpu.VMEM((C * XP,), F32),
        pltpu.VMEM((C * XP,), F32),
        pltpu.SemaphoreType.DMA,
        pltpu.SemaphoreType.DMA,
        pltpu.SemaphoreType.DMA,
        pltpu.SemaphoreType.DMA,
        pltpu.SemaphoreType.DMA,
        pltpu.SemaphoreType.DMA,
    ]

    def body(a_h, b_h, x_h, row_h, col_h, ag_h, bg_h, xd_h,
             xpv, rowc0, rowc1, colc0, colc1, bufa0, bufa1, bufb0, bufb1,
             bufd0, bufd1, isem0, isem1, gsem0, gsem1, wsem0, wsem1):
        wid = lax.axis_index("s") * NC + lax.axis_index("c")
        pltpu.sync_copy(x_h, xpv)
        iota = lax.iota(jnp.int32, 16)
        rowc = (rowc0, rowc1)
        colc = (colc0, colc1)
        bufa = (bufa0, bufa1)
        bufb = (bufb0, bufb1)
        bufd = (bufd0, bufd1)
        isems = (isem0, isem1)
        gsems = (gsem0, gsem1)
        wsems = (wsem0, wsem1)

        def fire_idx(j, s):
            base = wid * epw + j * C
            return (pltpu.async_copy(row_h.at[pl.ds(base, C)], rowc[s],
                                     isems[s]),
                    pltpu.async_copy(col_h.at[pl.ds(base, C)], colc[s],
                                     isems[s]))

        def fire_gathers(s):
            return (pltpu.async_copy(a_h.at[rowc[s]], bufa[s], gsems[s]),
                    pltpu.async_copy(b_h.at[colc[s]], bufb[s], gsems[s]))

        def compute_xd(s):
            for k in range(ngrp):
                ridx = rowc[s][pl.ds(k * 16, 16)] * XP
                cidx = colc[s][pl.ds(k * 16, 16)] * XP
                eidx = (iota + k * 16) * XP
                rad = jnp.zeros((16,), F32)
                for l in range(3):
                    dl = (plsc.load_gather(xpv, [ridx + l])
                          - plsc.load_gather(xpv, [cidx + l]))
                    plsc.store_scatter(bufd[s], [eidx + l], dl)
                    rad = rad + dl * dl
                plsc.store_scatter(bufd[s], [eidx + 3], rad)

        def fire_writes(j, s):
            base = wid * epw + j * C
            return (
                pltpu.async_copy(bufa[s], ag_h.at[pl.ds(base, C)], wsems[s]),
                pltpu.async_copy(bufb[s], bg_h.at[pl.ds(base, C)], wsems[s]),
                pltpu.async_copy(bufd[s],
                                 xd_h.at[pl.ds(base * XP, C * XP)], wsems[s]),
            )

        @pl.loop(0, nch // 2)
        def _pair(t):
            j0 = 2 * t
            j1 = j0 + 1
            i0 = fire_idx(j0, 0)
            i1 = fire_idx(j1, 1)
            i0[0].wait()
            i0[1].wait()
            g0 = fire_gathers(0)
            i1[0].wait()
            i1[1].wait()
            g1 = fire_gathers(1)
            compute_xd(0)
            g0[0].wait()
            g0[1].wait()
            w0 = fire_writes(j0, 0)
            compute_xd(1)
            g1[0].wait()
            g1[1].wait()
            w1 = fire_writes(j1, 1)
            for w in w0 + w1:
                w.wait()

        if nch % 2:
            @pl.loop(nch - 1, nch)
            def _tail(j):
                i0 = fire_idx(j, 0)
                i0[0].wait()
                i0[1].wait()
                g0 = fire_gathers(0)
                compute_xd(0)
                g0[0].wait()
                g0[1].wait()
                w0 = fire_writes(j, 0)
                for w in w0:
                    w.wait()

    return pl.kernel(body, out_type=out_type, mesh=mesh,
                     scratch_types=scratch,
                     compiler_params=pltpu.CompilerParams(
                         needs_layout_passes=False),
                     )(A, B, xp, row, col)


def _sc_scatter(m, tr, row, zh, zx, E):
    epw = E // NW
    nch = epw // C
    rows_per_tile = NP_PAD // NS
    xwords_per_tile = NP_PAD * XP // NS
    nxs = C * XP // 80
    mesh = plsc.VectorSubcoreMesh(core_axis_name="c", subcore_axis_name="s")
    out_type = (
        jax.ShapeDtypeStruct((NC, NP_PAD, H), F32),
        jax.ShapeDtypeStruct((NC, NP_PAD * XP), F32),
    )
    scratch = [
        pltpu.VMEM((C,), jnp.int32),
        pltpu.VMEM((C,), jnp.int32),
        pltpu.VMEM((C, H), F32),
        pltpu.VMEM((C, H), F32),
        pltpu.VMEM((C * XP,), F32),
        pltpu.VMEM((C * XP,), F32),
        pltpu.VMEM((nxs, 80), jnp.int32),
        pltpu.VMEM((nxs, 80), jnp.int32),
        pltpu.VMEM_SHARED((NP_PAD, H), F32),
        pltpu.VMEM_SHARED((NP_PAD * XP,), F32),
        pltpu.SemaphoreType.DMA,
        pltpu.SemaphoreType.DMA,
        pltpu.SemaphoreType.DMA,
        pltpu.SemaphoreType.DMA,
    ]

    def body(m_h, tr_h, row_h, zh_h, zx_h, aggh_h, aggx_h,
             rowc0, rowc1, bufm0, bufm1, buft0, buft1, idxb0, idxb1,
             sh, sx, lsem0, lsem1, asem0, asem1):
        rowc = (rowc0, rowc1)
        bufm = (bufm0, bufm1)
        buft = (buft0, buft1)
        idxb = (idxb0, idxb1)
        cid = lax.axis_index("c")
        sid = lax.axis_index("s")
        wid = sid * NC + cid
        r0 = sid * rows_per_tile
        x0 = sid * xwords_per_tile
        pltpu.sync_copy(zx_h, sx.at[pl.ds(x0, xwords_per_tile)])
        pltpu.sync_copy(zh_h, sh.at[pl.ds(r0, rows_per_tile)])
        plsc.subcore_barrier()
        iota = lax.iota(jnp.int32, 16)
        imod = jnp.bitwise_and(iota, XP - 1)
        ediv = jnp.right_shift(iota, 2)
        lsems = (lsem0, lsem1)
        asems = (asem0, asem1)

        def load_chunk(j, s):
            base = wid * epw + j * C
            return (
                pltpu.async_copy(m_h.at[pl.ds(base, C)], bufm[s], lsems[s]),
                pltpu.async_copy(tr_h.at[pl.ds(base * XP, C * XP)],
                                 buft[s], lsems[s]),
                pltpu.async_copy(row_h.at[pl.ds(base, C)], rowc[s],
                                 lsems[s]),
            )

        def do_adds(s):
            dm = pltpu.async_copy(bufm[s], sh.at[rowc[s]], asems[s],
                                  add=True)
            for st in range(nxs):
                for v in range(5):
                    e0 = st * 20 + v * 4
                    rv = plsc.load_gather(rowc[s], [e0 + ediv])
                    idxb[s][st, pl.ds(v * 16, 16)] = rv * XP + imod
            outs = [dm]
            for st in range(nxs):
                outs.append(pltpu.async_copy(
                    buft[s].at[pl.ds(st * 80, 80)],
                    sx.at[idxb[s].at[st]], asems[s], add=True))
            return outs

        @pl.loop(0, nch // 2)
        def _pair(t):
            j0 = 2 * t
            j1 = j0 + 1
            l0 = load_chunk(j0, 0)
            l1 = load_chunk(j1, 1)
            for d in l0:
                d.wait()
            a0 = do_adds(0)
            for d in l1:
                d.wait()
            a1 = do_adds(1)
            for d in a0:
                d.wait()
            for d in a1:
                d.wait()

        if nch % 2:
            @pl.loop(nch - 1, nch)
            def _tail(j):
                l0 = load_chunk(j, 0)
                for d in l0:
                    d.wait()
                for d in do_adds(0):
                    d.wait()

        plsc.subcore_barrier()
        pltpu.sync_copy(sh.at[pl.ds(r0, rows_per_tile)],
                        aggh_h.at[cid, pl.ds(r0, rows_per_tile)])
        pltpu.sync_copy(sx.at[pl.ds(x0, xwords_per_tile)],
                        aggx_h.at[cid, pl.ds(x0, xwords_per_tile)])

    return pl.kernel(body, out_type=out_type, mesh=mesh,
                     scratch_types=scratch,
                     compiler_params=pltpu.CompilerParams(
                         needs_layout_passes=False),
                     )(m, tr, row, zh, zx)


def _full_spec(shape):
    nd = len(shape)
    return pl.BlockSpec(shape, lambda i: (0,) * nd)


def _edge_block(E):
    return max(b for b in range(8, min(2560, E) + 1, 8) if E % b == 0)


def _tc_edge(ag, bg, xd, ea, wr, we1e, we2, be2, wc1, bc1, wc2p):
    E = ag.shape[0]
    BE = _edge_block(E)
    DE = ea.shape[1]

    def body(ag_r, bg_r, xd_r, ea_r, wr_r, we1e_r, we2_r, be2_r,
             wc1_r, bc1_r, wc2_r, m_r, tr_r):
        lane3 = _lane3()
        mask012 = _mask012()
        d = xd_r[...] * mask012
        radial = jnp.sum(xd_r[...] * lane3, axis=1, keepdims=True)
        pre = (ag_r[...] + bg_r[...] + radial * wr_r[...]
               + jnp.dot(ea_r[...], we1e_r[...], preferred_element_type=F32))
        m1 = _silu(pre)
        m2 = _silu(jnp.dot(m1, we2_r[...], preferred_element_type=F32)
                   + be2_r[...])
        u = _silu(jnp.dot(m2, wc1_r[...], preferred_element_type=F32)
                  + bc1_r[...])
        w8 = jnp.dot(u, wc2_r[...], preferred_element_type=F32)
        w = jnp.sum(w8, axis=1, keepdims=True)
        m_r[...] = m2
        tr_r[...] = d * w + lane3

    return pl.pallas_call(
        body,
        grid=(E // BE,),
        in_specs=[
            pl.BlockSpec((BE, H), lambda i: (i, 0)),
            pl.BlockSpec((BE, H), lambda i: (i, 0)),
            pl.BlockSpec((BE, XP), lambda i: (i, 0)),
            pl.BlockSpec((BE, DE), lambda i: (i, 0)),
            _full_spec((1, H)),
            _full_spec((DE, H)),
            _full_spec((H, H)),
            _full_spec((1, H)),
            _full_spec((H, H)),
            _full_spec((1, H)),
            _full_spec((H, XP)),
        ],
        out_specs=[
            pl.BlockSpec((BE, H), lambda i: (i, 0)),
            pl.BlockSpec((BE, XP), lambda i: (i, 0)),
        ],
        out_shape=[
            jax.ShapeDtypeStruct((E, H), F32),
            jax.ShapeDtypeStruct((E, XP), F32),
        ],
        compiler_params=pltpu.CompilerParams(
            dimension_semantics=("arbitrary",)),
    )(ag, bg, xd, ea, wr, we1e, we2, be2, wc1, bc1, wc2p)


def _node_block(N):
    return max(b for b in range(8, min(2048, N) + 1, 8) if N % b == 0)


def _tc_embed(h, wemb, bemb, we1r, be1, we1c):
    N, DIN = h.shape
    BN = _node_block(N)

    def body(h_r, wemb_r, bemb_r, we1r_r, be1_r, we1c_r, h0_r, a_r, b_r):
        h0 = jnp.dot(h_r[...], wemb_r[...], preferred_element_type=F32) \
            + bemb_r[...]
        h0_r[...] = h0
        a_r[...] = jnp.dot(h0, we1r_r[...], preferred_element_type=F32) \
            + be1_r[...]
        b_r[...] = jnp.dot(h0, we1c_r[...], preferred_element_type=F32)

    return pl.pallas_call(
        body,
        grid=(N // BN,),
        in_specs=[
            pl.BlockSpec((BN, DIN), lambda i: (i, 0)),
            _full_spec((DIN, H)),
            _full_spec((1, H)),
            _full_spec((H, H)),
            _full_spec((1, H)),
            _full_spec((H, H)),
        ],
        out_specs=[
            pl.BlockSpec((BN, H), lambda i: (i, 0)),
            pl.BlockSpec((BN, H), lambda i: (i, 0)),
            pl.BlockSpec((BN, H), lambda i: (i, 0)),
        ],
        out_shape=[
            jax.ShapeDtypeStruct((N, H), F32),
            jax.ShapeDtypeStruct((N, H), F32),
            jax.ShapeDtypeStruct((N, H), F32),
        ],
        compiler_params=pltpu.CompilerParams(
            dimension_semantics=("arbitrary",)),
    )(h, wemb, bemb, we1r, be1, we1c)


def _tc_node(h, xp, velp, aggh, aggx, wn1h, wn1a, bn1, wn2, bn2,
             wv1, bv1, wv2p, bv2p, we1rn, be1n, we1cn):
    N = h.shape[0]
    BN = _node_block(N)
    def body(h_r, xp_r, velp_r, aggh_r, aggx_r, wn1h_r, wn1a_r, bn1_r,
             wn2_r, bn2_r, wv1_r, bv1_r, wv2_r, bv2_r, we1rn_r, be1n_r,
             we1cn_r, ho_r, xo_r, a_r, b_r):
        lane3 = _lane3()
        mask012 = _mask012()
        hcur = h_r[...]
        ah = aggh_r[0] + aggh_r[1]
        ax = jnp.sum(aggx_r[...], axis=0)
        cnt = jnp.sum(ax * lane3, axis=1, keepdims=True)
        inv = COORDS_WEIGHT / jnp.maximum(cnt, 1.0)
        pv8 = jnp.dot(_silu(jnp.dot(hcur, wv1_r[...],
                                    preferred_element_type=F32) + bv1_r[...]),
                      wv2_r[...], preferred_element_type=F32) + bv2_r[...]
        phiv = jnp.sum(pv8, axis=1, keepdims=True)
        xo_r[...] = xp_r[...] + ax * mask012 * inv + phiv * velp_r[...]
        t = _silu(jnp.dot(hcur, wn1h_r[...], preferred_element_type=F32)
                  + jnp.dot(ah, wn1a_r[...], preferred_element_type=F32)
                  + bn1_r[...])
        hnew = hcur + jnp.dot(t, wn2_r[...], preferred_element_type=F32) \
            + bn2_r[...]
        ho_r[...] = hnew
        a_r[...] = jnp.dot(hnew, we1rn_r[...], preferred_element_type=F32) \
            + be1n_r[...]
        b_r[...] = jnp.dot(hnew, we1cn_r[...], preferred_element_type=F32)

    return pl.pallas_call(
        body,
        grid=(N // BN,),
        in_specs=[
            pl.BlockSpec((BN, H), lambda i: (i, 0)),
            pl.BlockSpec((BN, XP), lambda i: (i, 0)),
            pl.BlockSpec((BN, XP), lambda i: (i, 0)),
            pl.BlockSpec((NC, BN, H), lambda i: (0, i, 0)),
            pl.BlockSpec((NC, BN, XP), lambda i: (0, i, 0)),
            _full_spec((H, H)),
            _full_spec((H, H)),
            _full_spec((1, H)),
            _full_spec((H, H)),
            _full_spec((1, H)),
            _full_spec((H, H)),
            _full_spec((1, H)),
            _full_spec((H, XP)),
            _full_spec((1, XP)),
            _full_spec((H, H)),
            _full_spec((1, H)),
            _full_spec((H, H)),
        ],
        out_specs=[
            pl.BlockSpec((BN, H), lambda i: (i, 0)),
            pl.BlockSpec((BN, XP), lambda i: (i, 0)),
            pl.BlockSpec((BN, H), lambda i: (i, 0)),
            pl.BlockSpec((BN, H), lambda i: (i, 0)),
        ],
        out_shape=[
            jax.ShapeDtypeStruct((N, H), F32),
            jax.ShapeDtypeStruct((N, XP), F32),
            jax.ShapeDtypeStruct((N, H), F32),
            jax.ShapeDtypeStruct((N, H), F32),
        ],
        compiler_params=pltpu.CompilerParams(
            dimension_semantics=("arbitrary",)),
    )(h, xp, velp, aggh, aggx, wn1h, wn1a, bn1, wn2, bn2,
      wv1, bv1, wv2p, bv2p, we1rn, be1n, we1cn)


def _row2(v):
    return v.reshape(1, -1).astype(F32)


def _pad_minor(w, width):
    return jnp.pad(w.astype(F32), ((0, 0), (0, width - w.shape[1])))


def kernel(h, x, vel, edge_attr, params, edges):
    N = h.shape[0]
    E = edges.shape[1]
    row = edges[0]
    col = edges[1]

    xp = jnp.pad(x.astype(F32), ((0, 0), (0, XP - x.shape[1])))
    velp = jnp.pad(vel.astype(F32), ((0, 0), (0, XP - vel.shape[1])))
    zh = jnp.zeros((NP_PAD // NS, H), F32)
    zx = jnp.zeros((NP_PAD * XP // NS,), F32)

    wemb, bemb = params["emb"]
    lp = params["layers"]
    n_layers = len(lp)

    def we1_parts(p):
        we1 = p["We1"]
        return (we1[:H], we1[H:2 * H], _row2(we1[2 * H]),
                we1[2 * H + 1:], _row2(p["be1"]))

    h0, A, B = _tc_embed(h, wemb, _row2(bemb), *(
        lambda t: (t[0], t[4], t[1]))(we1_parts(lp[0])))

    hc, xc = h0, xp
    for l in range(n_layers):
        p = lp[l]
        _, _, wr, we1e, _ = we1_parts(p)
        ag, bg, xdf = _sc_gather(A, B, xc.reshape(-1), row, col, E)
        m, tr = _tc_edge(ag, bg, xdf.reshape(E, XP), edge_attr, wr, we1e,
                         p["We2"], _row2(p["be2"]), p["Wc1"],
                         _row2(p["bc1"]), _pad_minor(p["Wc2"], XP))
        aggh, aggxf = _sc_scatter(m, tr.reshape(-1), row, zh, zx, E)
        aggx = aggxf.reshape(NC, NP_PAD, XP)
        pn = lp[(l + 1) % n_layers]
        we1rn, we1cn, _, _, be1n = we1_parts(pn)
        wn1 = p["Wn1"]
        hc, xc, A, B = _tc_node(
            hc, xc, velp, aggh, aggx,
            wn1[:H], wn1[H:], _row2(p["bn1"]), p["Wn2"], _row2(p["bn2"]),
            p["Wv1"], _row2(p["bv1"]), _pad_minor(p["Wv2"], XP),
            _pad_minor(p["bv2"].reshape(1, 1), XP),
            we1rn, be1n, we1cn)

    return xc[:, :x.shape[1]]
